# baseline (speedup 1.0000x reference)
import functools

import jax
import jax.numpy as jnp
from jax import lax
from jax.experimental import pallas as pl
from jax.experimental.pallas import tpu as pltpu
from jax.experimental.pallas import tpu_sc as plsc

N = 10000
E = 320000
ND = 128
ED = 16
HID = 40
DP = 48
NC, NS = 2, 16
NW = NC * NS
BPW = E // NW
GCH = 1000
SCH = 80
EPS = 1e-5
F32 = jnp.float32


def _node_prep_body(x_ref, g_ref, b_ref, w1s_ref, w1d_ref, wq_ref, bq_ref,
                    ps_ref, pd_ref, q_ref):
    x = x_ref[...]
    mu = jnp.mean(x, axis=0, keepdims=True)
    xc = x - mu
    var = jnp.mean(xc * xc, axis=0, keepdims=True)
    xb = jnp.maximum(g_ref[...] * xc / jnp.sqrt(var + EPS) + b_ref[...], 0.0)
    ps_ref[...] = jnp.dot(xb, w1s_ref[...], preferred_element_type=F32)
    pd_ref[...] = jnp.dot(xb, w1d_ref[...], preferred_element_type=F32)
    q_ref[...] = jnp.dot(xb, wq_ref[...], preferred_element_type=F32) + bq_ref[...]


def _node_prep(cur_x, gamma, beta, w1s, w1d, wq, bq):
    return pl.pallas_call(
        _node_prep_body,
        out_shape=[
            jax.ShapeDtypeStruct((N, DP), F32),
            jax.ShapeDtypeStruct((N, DP), F32),
            jax.ShapeDtypeStruct((N, HID), F32),
        ],
    )(cur_x, gamma.reshape(1, ND), beta.reshape(1, ND), w1s, w1d, wq,
      bq.reshape(1, HID))


SBLK = 16000
NSB = E // SBLK


def _colstats_body(a_ref, o_ref, acc_ref):
    ph = pl.program_id(0)
    i = pl.program_id(1)

    @pl.when((ph == 0) & (i == 0))
    def _():
        acc_ref[...] = jnp.zeros_like(acc_ref)

    a = a_ref[...]

    @pl.when(ph == 0)
    def _():
        acc_ref[0:1, :] += jnp.sum(a, axis=0, keepdims=True)

    @pl.when(ph == 1)
    def _():
        mu = acc_ref[0:1, :] * (1.0 / E)
        acc_ref[1:2, :] += jnp.sum((a - mu) ** 2, axis=0, keepdims=True)

    @pl.when((ph == 1) & (i == NSB - 1))
    def _():
        o_ref[...] = acc_ref[...] * (1.0 / E)


def _colstats(arr):
    return pl.pallas_call(
        _colstats_body,
        grid=(2, NSB),
        in_specs=[pl.BlockSpec((SBLK, ED), lambda ph, i: (i, 0))],
        out_specs=pl.BlockSpec((2, ED), lambda ph, i: (0, 0)),
        out_shape=jax.ShapeDtypeStruct((2, ED), F32),
        scratch_shapes=[pltpu.VMEM((2, ED), F32)],
    )(arr)


BE = 8000


def _edge_body_mid(gs_ref, gd_ref, ea_ref, st_ref, gm_ref, bt_ref, w1e_ref,
                   b1_ref, w2_ref, b2_ref, eo_ref):
    mu = st_ref[0:1, :]
    var = st_ref[1:2, :]
    eb = jnp.maximum(
        gm_ref[...] * (ea_ref[...] - mu) / jnp.sqrt(var + EPS) + bt_ref[...], 0.0)
    c = jnp.dot(eb, w1e_ref[...], preferred_element_type=F32) + b1_ref[...]
    h = jnp.maximum(gs_ref[...][:, :HID] + gd_ref[...][:, :HID] + c, 0.0)
    eo_ref[...] = jnp.dot(h, w2_ref[...], preferred_element_type=F32) + b2_ref[...]


def _edge_body_fin(gs_ref, gd_ref, ea_ref, st_ref, gm_ref, bt_ref, w1e_ref,
                   b1_ref, w2_ref, b2_ref, eorig_ref, eo_ref, efin_ref):
    mu = st_ref[0:1, :]
    var = st_ref[1:2, :]
    eb = jnp.maximum(
        gm_ref[...] * (ea_ref[...] - mu) / jnp.sqrt(var + EPS) + bt_ref[...], 0.0)
    c = jnp.dot(eb, w1e_ref[...], preferred_element_type=F32) + b1_ref[...]
    h = jnp.maximum(gs_ref[...][:, :HID] + gd_ref[...][:, :HID] + c, 0.0)
    eo = jnp.dot(h, w2_ref[...], preferred_element_type=F32) + b2_ref[...]
    eo_ref[...] = eo
    efin_ref[...] = 0.5 * eorig_ref[...] + 0.5 * eo


def _edge_mlp(gs, gd, cur_e, stats, gamma_e, beta_e, w1e, b1, w2, b2, eorig):
    blk = lambda c: pl.BlockSpec((BE, c), lambda i: (i, 0))
    fix = lambda r, c: pl.BlockSpec((r, c), lambda i: (0, 0))
    args = [gs, gd, cur_e, stats, gamma_e.reshape(1, ED), beta_e.reshape(1, ED),
            w1e, b1.reshape(1, HID), w2, b2.reshape(1, ED)]
    in_specs = [blk(DP), blk(DP), blk(ED), fix(2, ED), fix(1, ED), fix(1, ED),
                fix(ED, HID), fix(1, HID), fix(HID, ED), fix(1, ED)]
    if eorig is None:
        return pl.pallas_call(
            _edge_body_mid,
            grid=(E // BE,),
            in_specs=in_specs,
            out_specs=blk(ED),
            out_shape=jax.ShapeDtypeStruct((E, ED), F32),
        )(*args)
    return pl.pallas_call(
        _edge_body_fin,
        grid=(E // BE,),
        in_specs=in_specs + [blk(ED)],
        out_specs=[blk(ED), blk(ED)],
        out_shape=[jax.ShapeDtypeStruct((E, ED), F32),
                   jax.ShapeDtypeStruct((E, ED), F32)],
    )(*(args + [eorig]))


def _node_body_mid(q_ref, agg_ref, wa_ref, w2_ref, b2_ref, out_ref):
    agg = agg_ref[0] + agg_ref[1]
    t = jnp.maximum(q_ref[...] + jnp.dot(agg, wa_ref[...],
                                         preferred_element_type=F32), 0.0)
    out_ref[...] = jnp.dot(t, w2_ref[...], preferred_element_type=F32) + b2_ref[...]


def _node_body_fin(q_ref, agg_ref, wa_ref, w2_ref, b2_ref, xorig_ref, out_ref):
    agg = agg_ref[0] + agg_ref[1]
    t = jnp.maximum(q_ref[...] + jnp.dot(agg, wa_ref[...],
                                         preferred_element_type=F32), 0.0)
    xo = jnp.dot(t, w2_ref[...], preferred_element_type=F32) + b2_ref[...]
    out_ref[...] = 0.5 * xorig_ref[...] + 0.5 * xo


def _node_mlp(q, agg2, wa, w2, b2, xorig):
    args = [q, agg2, wa, w2, b2.reshape(1, ND)]
    if xorig is None:
        return pl.pallas_call(
            _node_body_mid,
            out_shape=jax.ShapeDtypeStruct((N, ND), F32),
        )(*args)
    return pl.pallas_call(
        _node_body_fin,
        out_shape=jax.ShapeDtypeStruct((N, ND), F32),
    )(*(args + [xorig]))


def _sc_gather(ps, pd, src, dst):
    mesh = plsc.VectorSubcoreMesh(core_axis_name="c", subcore_axis_name="s")

    @functools.partial(
        pl.kernel, mesh=mesh,
        out_type=[jax.ShapeDtypeStruct((E, DP), F32),
                  jax.ShapeDtypeStruct((E, DP), F32)],
        compiler_params=pltpu.CompilerParams(use_tc_tiling_on_sc=False),
        scratch_types=[
            pltpu.VMEM((GCH,), jnp.int32),
            pltpu.VMEM((GCH,), jnp.int32),
            pltpu.VMEM((GCH, DP), F32),
            pltpu.VMEM((GCH, DP), F32),
            pltpu.SemaphoreType.DMA,
            pltpu.SemaphoreType.DMA,
        ],
    )
    def k(ps_hbm, pd_hbm, src_hbm, dst_hbm, outs_hbm, outd_hbm,
          si, di, rs, rd, s1, s2):
        wid = lax.axis_index("s") * NC + lax.axis_index("c")
        base = wid * BPW

        def chunk(j, carry):
            off = base + j * GCH
            pltpu.sync_copy(src_hbm.at[pl.ds(off, GCH)], si)
            pltpu.sync_copy(dst_hbm.at[pl.ds(off, GCH)], di)
            c1 = pltpu.async_copy(ps_hbm.at[si], rs, s1)
            c2 = pltpu.async_copy(pd_hbm.at[di], rd, s2)
            c1.wait()
            pltpu.sync_copy(rs, outs_hbm.at[pl.ds(off, GCH)])
            c2.wait()
            pltpu.sync_copy(rd, outd_hbm.at[pl.ds(off, GCH)])
            return carry

        lax.fori_loop(0, BPW // GCH, chunk, 0)

    return k(ps, pd, src, dst)


def _sc_scatter(eo, dst):
    mesh = plsc.VectorSubcoreMesh(core_axis_name="c", subcore_axis_name="s")
    rpt = N // NS

    @functools.partial(
        pl.kernel, mesh=mesh,
        out_type=jax.ShapeDtypeStruct((NC, N, ED), F32),
        compiler_params=pltpu.CompilerParams(use_tc_tiling_on_sc=False),
        scratch_types=[
            pltpu.VMEM((SCH,), jnp.int32),
            pltpu.VMEM((SCH, ED), F32),
            pltpu.VMEM((rpt, ED), F32),
            pltpu.VMEM_SHARED((N, ED), F32),
        ],
    )
    def k(eo_hbm, dst_hbm, out_hbm, idxv, rows, stage, aggsh):
        cid = lax.axis_index("c")
        sid = lax.axis_index("s")
        wid = sid * NC + cid
        base = wid * BPW

        def zrow(r, carry):
            stage[r, pl.ds(0, ED)] = jnp.zeros((ED,), F32)
            return carry

        lax.fori_loop(0, rpt, zrow, 0)
        pltpu.sync_copy(stage, aggsh.at[pl.ds(sid * rpt, rpt)])
        plsc.subcore_barrier()

        def chunk(j, carry):
            off = base + j * SCH
            pltpu.sync_copy(dst_hbm.at[pl.ds(off, SCH)], idxv)
            pltpu.sync_copy(eo_hbm.at[pl.ds(off, SCH)], rows)
            pltpu.sync_copy(rows, aggsh.at[idxv], add=True)
            return carry

        lax.fori_loop(0, BPW // SCH, chunk, 0)
        plsc.subcore_barrier()
        pltpu.sync_copy(aggsh.at[pl.ds(sid * rpt, rpt)], stage)
        pltpu.sync_copy(stage, out_hbm.at[cid, pl.ds(sid * rpt, rpt)])

    return k(eo, dst)


def kernel(x, edge_index, edge_attr, params):
    layers = params["layers"]
    src = edge_index[0]
    dst = edge_index[1]
    cur_x, cur_e = x, edge_attr
    out_x = out_e = None
    for li, p in enumerate(layers):
        final = li == len(layers) - 1
        pe, pn = p["edge_mlp"], p["node_mlp"]
        w1 = pe["W1"]
        pad = ((0, 0), (0, DP - HID))
        w1s = jnp.pad(w1[:ND], pad)
        w1d = jnp.pad(w1[ND:2 * ND], pad)
        w1e = w1[2 * ND:]
        wn1 = pn["W1"]
        ps, pd_, q = _node_prep(cur_x, p["bn_node"]["gamma"], p["bn_node"]["beta"],
                                w1s, w1d, wn1[:ND], pn["b1"])
        stats = _colstats(cur_e)
        gs, gd = _sc_gather(ps, pd_, src, dst)
        if final:
            eo, out_e = _edge_mlp(gs, gd, cur_e, stats, p["bn_edge"]["gamma"],
                                  p["bn_edge"]["beta"], w1e, pe["b1"], pe["W2"],
                                  pe["b2"], edge_attr)
        else:
            eo = _edge_mlp(gs, gd, cur_e, stats, p["bn_edge"]["gamma"],
                           p["bn_edge"]["beta"], w1e, pe["b1"], pe["W2"],
                           pe["b2"], None)
        agg2 = _sc_scatter(eo, dst)
        xo = _node_mlp(q, agg2, wn1[ND:], pn["W2"], pn["b2"],
                       x if final else None)
        if final:
            out_x = xo
        cur_x, cur_e = xo, eo
    return (out_x, out_e)


# speedup vs baseline: 1.2191x; 1.2191x over previous
import functools

import jax
import jax.numpy as jnp
from jax import lax
from jax.experimental import pallas as pl
from jax.experimental.pallas import tpu as pltpu
from jax.experimental.pallas import tpu_sc as plsc

N = 10000
E = 320000
ND = 128
ED = 16
HID = 40
DP = 48
NC, NS = 2, 16
NW = NC * NS
BPW = E // NW
GCH = 400
SCH = 80
EPS = 1e-5
F32 = jnp.float32


def _node_prep_body(x_ref, g_ref, b_ref, w1s_ref, w1d_ref, wq_ref, bq_ref,
                    ps_ref, pd_ref, q_ref):
    x = x_ref[...]
    mu = jnp.mean(x, axis=0, keepdims=True)
    xc = x - mu
    var = jnp.mean(xc * xc, axis=0, keepdims=True)
    xb = jnp.maximum(g_ref[...] * xc / jnp.sqrt(var + EPS) + b_ref[...], 0.0)
    ps_ref[...] = jnp.dot(xb, w1s_ref[...], preferred_element_type=F32)
    pd_ref[...] = jnp.dot(xb, w1d_ref[...], preferred_element_type=F32)
    q_ref[...] = jnp.dot(xb, wq_ref[...], preferred_element_type=F32) + bq_ref[...]


def _node_prep(cur_x, gamma, beta, w1s, w1d, wq, bq):
    return pl.pallas_call(
        _node_prep_body,
        out_shape=[
            jax.ShapeDtypeStruct((N, DP), F32),
            jax.ShapeDtypeStruct((N, DP), F32),
            jax.ShapeDtypeStruct((N, HID), F32),
        ],
    )(cur_x, gamma.reshape(1, ND), beta.reshape(1, ND), w1s, w1d, wq,
      bq.reshape(1, HID))


SBLK = 16000
NSB = E // SBLK


def _colstats_body(a_ref, o_ref, acc_ref):
    ph = pl.program_id(0)
    i = pl.program_id(1)

    @pl.when((ph == 0) & (i == 0))
    def _():
        acc_ref[...] = jnp.zeros_like(acc_ref)

    a = a_ref[...]

    @pl.when(ph == 0)
    def _():
        acc_ref[0:1, :] += jnp.sum(a, axis=0, keepdims=True)

    @pl.when(ph == 1)
    def _():
        mu = acc_ref[0:1, :] * (1.0 / E)
        acc_ref[1:2, :] += jnp.sum((a - mu) ** 2, axis=0, keepdims=True)

    @pl.when((ph == 1) & (i == NSB - 1))
    def _():
        o_ref[...] = acc_ref[...] * (1.0 / E)


def _colstats(arr):
    return pl.pallas_call(
        _colstats_body,
        grid=(2, NSB),
        in_specs=[pl.BlockSpec((SBLK, ED), lambda ph, i: (i, 0))],
        out_specs=pl.BlockSpec((2, ED), lambda ph, i: (0, 0)),
        out_shape=jax.ShapeDtypeStruct((2, ED), F32),
        scratch_shapes=[pltpu.VMEM((2, ED), F32)],
    )(arr)


BE = 8000


def _edge_body_mid(g_ref, ea_ref, st_ref, gm_ref, bt_ref, w1e_ref, b1_ref,
                   w2_ref, b2_ref, eo_ref):
    mu = st_ref[0:1, :]
    var = st_ref[1:2, :]
    eb = jnp.maximum(
        gm_ref[...] * (ea_ref[...] - mu) / jnp.sqrt(var + EPS) + bt_ref[...], 0.0)
    c = jnp.dot(eb, w1e_ref[...], preferred_element_type=F32) + b1_ref[...]
    h = jnp.maximum(g_ref[...][:, :HID] + c, 0.0)
    eo_ref[...] = jnp.dot(h, w2_ref[...], preferred_element_type=F32) + b2_ref[...]


def _edge_body_fin(g_ref, ea_ref, st_ref, gm_ref, bt_ref, w1e_ref, b1_ref,
                   w2_ref, b2_ref, eorig_ref, eo_ref, efin_ref):
    mu = st_ref[0:1, :]
    var = st_ref[1:2, :]
    eb = jnp.maximum(
        gm_ref[...] * (ea_ref[...] - mu) / jnp.sqrt(var + EPS) + bt_ref[...], 0.0)
    c = jnp.dot(eb, w1e_ref[...], preferred_element_type=F32) + b1_ref[...]
    h = jnp.maximum(g_ref[...][:, :HID] + c, 0.0)
    eo = jnp.dot(h, w2_ref[...], preferred_element_type=F32) + b2_ref[...]
    eo_ref[...] = eo
    efin_ref[...] = 0.5 * eorig_ref[...] + 0.5 * eo


def _edge_mlp(g, cur_e, stats, gamma_e, beta_e, w1e, b1, w2, b2, eorig):
    blk = lambda c: pl.BlockSpec((BE, c), lambda i: (i, 0))
    fix = lambda r, c: pl.BlockSpec((r, c), lambda i: (0, 0))
    args = [g, cur_e, stats, gamma_e.reshape(1, ED), beta_e.reshape(1, ED),
            w1e, b1.reshape(1, HID), w2, b2.reshape(1, ED)]
    in_specs = [blk(DP), blk(ED), fix(2, ED), fix(1, ED), fix(1, ED),
                fix(ED, HID), fix(1, HID), fix(HID, ED), fix(1, ED)]
    if eorig is None:
        return pl.pallas_call(
            _edge_body_mid,
            grid=(E // BE,),
            in_specs=in_specs,
            out_specs=blk(ED),
            out_shape=jax.ShapeDtypeStruct((E, ED), F32),
        )(*args)
    return pl.pallas_call(
        _edge_body_fin,
        grid=(E // BE,),
        in_specs=in_specs + [blk(ED)],
        out_specs=[blk(ED), blk(ED)],
        out_shape=[jax.ShapeDtypeStruct((E, ED), F32),
                   jax.ShapeDtypeStruct((E, ED), F32)],
    )(*(args + [eorig]))


def _node_body_mid(q_ref, agg_ref, wa_ref, w2_ref, b2_ref, out_ref):
    agg = agg_ref[0] + agg_ref[1]
    t = jnp.maximum(q_ref[...] + jnp.dot(agg, wa_ref[...],
                                         preferred_element_type=F32), 0.0)
    out_ref[...] = jnp.dot(t, w2_ref[...], preferred_element_type=F32) + b2_ref[...]


def _node_body_fin(q_ref, agg_ref, wa_ref, w2_ref, b2_ref, xorig_ref, out_ref):
    agg = agg_ref[0] + agg_ref[1]
    t = jnp.maximum(q_ref[...] + jnp.dot(agg, wa_ref[...],
                                         preferred_element_type=F32), 0.0)
    xo = jnp.dot(t, w2_ref[...], preferred_element_type=F32) + b2_ref[...]
    out_ref[...] = 0.5 * xorig_ref[...] + 0.5 * xo


def _node_mlp(q, agg2, wa, w2, b2, xorig):
    args = [q, agg2, wa, w2, b2.reshape(1, ND)]
    if xorig is None:
        return pl.pallas_call(
            _node_body_mid,
            out_shape=jax.ShapeDtypeStruct((N, ND), F32),
        )(*args)
    return pl.pallas_call(
        _node_body_fin,
        out_shape=jax.ShapeDtypeStruct((N, ND), F32),
    )(*(args + [xorig]))


GNB = 2
GNC = BPW // GCH


def _sc_gather(ps, pd, src, dst):
    mesh = plsc.VectorSubcoreMesh(core_axis_name="c", subcore_axis_name="s")

    @functools.partial(
        pl.kernel, mesh=mesh,
        out_type=jax.ShapeDtypeStruct((E, DP), F32),
        compiler_params=pltpu.CompilerParams(use_tc_tiling_on_sc=False),
        scratch_types=[
            pltpu.VMEM((BPW,), jnp.int32),
            pltpu.VMEM((BPW,), jnp.int32),
        ] + [pltpu.VMEM((GCH, DP), F32) for _ in range(2 * GNB)]
          + [pltpu.SemaphoreType.DMA for _ in range(2 * GNB)],
    )
    def k(ps_hbm, pd_hbm, src_hbm, dst_hbm, out_hbm, si_all, di_all, *bufs):
        rs = bufs[0:GNB]
        rd = bufs[GNB:2 * GNB]
        sg = bufs[2 * GNB:3 * GNB]
        ss = bufs[3 * GNB:4 * GNB]
        wid = lax.axis_index("s") * NC + lax.axis_index("c")
        base = wid * BPW
        pltpu.sync_copy(src_hbm.at[pl.ds(base, BPW)], si_all)
        pltpu.sync_copy(dst_hbm.at[pl.ds(base, BPW)], di_all)

        def issue_gather(j, b):
            loc = j * GCH
            c1 = pltpu.async_copy(ps_hbm.at[si_all.at[pl.ds(loc, GCH)]],
                                  rs[b], sg[b])
            c2 = pltpu.async_copy(pd_hbm.at[di_all.at[pl.ds(loc, GCH)]],
                                  rd[b], sg[b])
            return c1, c2

        inflight = [None] * GNB
        stores = [None] * GNB
        for j in range(min(GNB, GNC)):
            inflight[j % GNB] = issue_gather(j, j % GNB)
        for j in range(GNC):
            b = j % GNB
            c1, c2 = inflight[b]
            c1.wait()
            c2.wait()

            def addrow(r, cc):
                for c in range(DP // 16):
                    sl = (r, pl.ds(c * 16, 16))
                    rs[b][sl] = rs[b][sl] + rd[b][sl]
                return cc

            lax.fori_loop(0, GCH, addrow, 0)
            stores[b] = pltpu.async_copy(
                rs[b], out_hbm.at[pl.ds(base + j * GCH, GCH)], ss[b])
            if j + GNB < GNC:
                stores[b].wait()
                inflight[b] = issue_gather(j + GNB, b)
        for j in range(max(GNC - GNB, 0), GNC):
            stores[j % GNB].wait()

    return k(ps, pd, src, dst)


def _sc_scatter(eo, dst):
    mesh = plsc.VectorSubcoreMesh(core_axis_name="c", subcore_axis_name="s")
    rpt = N // NS

    @functools.partial(
        pl.kernel, mesh=mesh,
        out_type=jax.ShapeDtypeStruct((NC, N, ED), F32),
        compiler_params=pltpu.CompilerParams(use_tc_tiling_on_sc=False),
        scratch_types=[
            pltpu.VMEM((SCH,), jnp.int32),
            pltpu.VMEM((SCH, ED), F32),
            pltpu.VMEM((rpt, ED), F32),
            pltpu.VMEM_SHARED((N, ED), F32),
        ],
    )
    def k(eo_hbm, dst_hbm, out_hbm, idxv, rows, stage, aggsh):
        cid = lax.axis_index("c")
        sid = lax.axis_index("s")
        wid = sid * NC + cid
        base = wid * BPW

        def zrow(r, carry):
            stage[r, pl.ds(0, ED)] = jnp.zeros((ED,), F32)
            return carry

        lax.fori_loop(0, rpt, zrow, 0)
        pltpu.sync_copy(stage, aggsh.at[pl.ds(sid * rpt, rpt)])
        plsc.subcore_barrier()

        def chunk(j, carry):
            off = base + j * SCH
            pltpu.sync_copy(dst_hbm.at[pl.ds(off, SCH)], idxv)
            pltpu.sync_copy(eo_hbm.at[pl.ds(off, SCH)], rows)
            pltpu.sync_copy(rows, aggsh.at[idxv], add=True)
            return carry

        lax.fori_loop(0, BPW // SCH, chunk, 0)
        plsc.subcore_barrier()
        pltpu.sync_copy(aggsh.at[pl.ds(sid * rpt, rpt)], stage)
        pltpu.sync_copy(stage, out_hbm.at[cid, pl.ds(sid * rpt, rpt)])

    return k(eo, dst)


def kernel(x, edge_index, edge_attr, params):
    layers = params["layers"]
    src = edge_index[0]
    dst = edge_index[1]
    cur_x, cur_e = x, edge_attr
    out_x = out_e = None
    for li, p in enumerate(layers):
        final = li == len(layers) - 1
        pe, pn = p["edge_mlp"], p["node_mlp"]
        w1 = pe["W1"]
        pad = ((0, 0), (0, DP - HID))
        w1s = jnp.pad(w1[:ND], pad)
        w1d = jnp.pad(w1[ND:2 * ND], pad)
        w1e = w1[2 * ND:]
        wn1 = pn["W1"]
        ps, pd_, q = _node_prep(cur_x, p["bn_node"]["gamma"], p["bn_node"]["beta"],
                                w1s, w1d, wn1[:ND], pn["b1"])
        stats = _colstats(cur_e)
        g = _sc_gather(ps, pd_, src, dst)
        if final:
            eo, out_e = _edge_mlp(g, cur_e, stats, p["bn_edge"]["gamma"],
                                  p["bn_edge"]["beta"], w1e, pe["b1"], pe["W2"],
                                  pe["b2"], edge_attr)
        else:
            eo = _edge_mlp(g, cur_e, stats, p["bn_edge"]["gamma"],
                           p["bn_edge"]["beta"], w1e, pe["b1"], pe["W2"],
                           pe["b2"], None)
        agg2 = _sc_scatter(eo, dst)
        xo = _node_mlp(q, agg2, wn1[ND:], pn["W2"], pn["b2"],
                       x if final else None)
        if final:
            out_x = xo
        cur_x, cur_e = xo, eo
    return (out_x, out_e)


# speedup vs baseline: 1.5598x; 1.2795x over previous
import functools

import jax
import jax.numpy as jnp
from jax import lax
from jax.experimental import pallas as pl
from jax.experimental.pallas import tpu as pltpu
from jax.experimental.pallas import tpu_sc as plsc

N = 10000
E = 320000
ND = 128
ED = 16
HID = 40
DP = 128
ADD_LANES = 48
NC, NS = 2, 16
NW = NC * NS
BPW = E // NW
GCH = 200
SCH = 100
EPS = 1e-5
F32 = jnp.float32


def _node_prep_body(x_ref, g_ref, b_ref, w1s_ref, w1d_ref, wq_ref, bq_ref,
                    ps_ref, pd_ref, q_ref):
    x = x_ref[...]
    mu = jnp.mean(x, axis=0, keepdims=True)
    xc = x - mu
    var = jnp.mean(xc * xc, axis=0, keepdims=True)
    xb = jnp.maximum(g_ref[...] * xc / jnp.sqrt(var + EPS) + b_ref[...], 0.0)
    ps_ref[...] = jnp.dot(xb, w1s_ref[...], preferred_element_type=F32)
    pd_ref[...] = jnp.dot(xb, w1d_ref[...], preferred_element_type=F32)
    q_ref[...] = jnp.dot(xb, wq_ref[...], preferred_element_type=F32) + bq_ref[...]


def _node_prep(cur_x, gamma, beta, w1s, w1d, wq, bq):
    return pl.pallas_call(
        _node_prep_body,
        out_shape=[
            jax.ShapeDtypeStruct((N, DP), F32),
            jax.ShapeDtypeStruct((N, DP), F32),
            jax.ShapeDtypeStruct((N, HID), F32),
        ],
    )(cur_x, gamma.reshape(1, ND), beta.reshape(1, ND), w1s, w1d, wq,
      bq.reshape(1, HID))


SBLK = 16000
NSB = E // SBLK


def _colstats_body(a_ref, o_ref, acc_ref):
    i = pl.program_id(0)

    @pl.when(i == 0)
    def _():
        acc_ref[...] = jnp.zeros_like(acc_ref)

    a = a_ref[...]
    acc_ref[0:1, :] += jnp.sum(a, axis=0, keepdims=True)
    acc_ref[1:2, :] += jnp.sum(a * a, axis=0, keepdims=True)

    @pl.when(i == NSB - 1)
    def _():
        mu = acc_ref[0:1, :] * (1.0 / E)
        var = acc_ref[1:2, :] * (1.0 / E) - mu * mu
        o_ref[...] = jnp.concatenate([mu, var], axis=0)


def _colstats(arr):
    return pl.pallas_call(
        _colstats_body,
        grid=(NSB,),
        in_specs=[pl.BlockSpec((SBLK, ED), lambda i: (i, 0))],
        out_specs=pl.BlockSpec((2, ED), lambda i: (0, 0)),
        out_shape=jax.ShapeDtypeStruct((2, ED), F32),
        scratch_shapes=[pltpu.VMEM((2, ED), F32)],
    )(arr)


BE = 8000


def _edge_body_mid(g_ref, ea_ref, st_ref, gm_ref, bt_ref, w1e_ref, b1_ref,
                   w2_ref, b2_ref, eo_ref):
    mu = st_ref[0:1, :]
    var = st_ref[1:2, :]
    eb = jnp.maximum(
        gm_ref[...] * (ea_ref[...] - mu) / jnp.sqrt(var + EPS) + bt_ref[...], 0.0)
    c = jnp.dot(eb, w1e_ref[...], preferred_element_type=F32) + b1_ref[...]
    h = jnp.maximum(g_ref[...][:, :HID] + c, 0.0)
    eo_ref[...] = jnp.dot(h, w2_ref[...], preferred_element_type=F32) + b2_ref[...]


def _edge_body_fin(g_ref, ea_ref, st_ref, gm_ref, bt_ref, w1e_ref, b1_ref,
                   w2_ref, b2_ref, eorig_ref, eo_ref, efin_ref):
    mu = st_ref[0:1, :]
    var = st_ref[1:2, :]
    eb = jnp.maximum(
        gm_ref[...] * (ea_ref[...] - mu) / jnp.sqrt(var + EPS) + bt_ref[...], 0.0)
    c = jnp.dot(eb, w1e_ref[...], preferred_element_type=F32) + b1_ref[...]
    h = jnp.maximum(g_ref[...][:, :HID] + c, 0.0)
    eo = jnp.dot(h, w2_ref[...], preferred_element_type=F32) + b2_ref[...]
    eo_ref[...] = eo
    efin_ref[...] = 0.5 * eorig_ref[...] + 0.5 * eo


def _edge_mlp(g, cur_e, stats, gamma_e, beta_e, w1e, b1, w2, b2, eorig):
    blk = lambda c: pl.BlockSpec((BE, c), lambda i: (i, 0))
    fix = lambda r, c: pl.BlockSpec((r, c), lambda i: (0, 0))
    args = [g, cur_e, stats, gamma_e.reshape(1, ED), beta_e.reshape(1, ED),
            w1e, b1.reshape(1, HID), w2, b2.reshape(1, ED)]
    in_specs = [blk(DP), blk(ED), fix(2, ED), fix(1, ED), fix(1, ED),
                fix(ED, HID), fix(1, HID), fix(HID, ED), fix(1, ED)]
    if eorig is None:
        return pl.pallas_call(
            _edge_body_mid,
            grid=(E // BE,),
            in_specs=in_specs,
            out_specs=blk(ED),
            out_shape=jax.ShapeDtypeStruct((E, ED), F32),
        )(*args)
    return pl.pallas_call(
        _edge_body_fin,
        grid=(E // BE,),
        in_specs=in_specs + [blk(ED)],
        out_specs=[blk(ED), blk(ED)],
        out_shape=[jax.ShapeDtypeStruct((E, ED), F32),
                   jax.ShapeDtypeStruct((E, ED), F32)],
    )(*(args + [eorig]))


def _node_body_mid(q_ref, agg_ref, wa_ref, w2_ref, b2_ref, out_ref):
    agg = agg_ref[0] + agg_ref[1]
    t = jnp.maximum(q_ref[...] + jnp.dot(agg, wa_ref[...],
                                         preferred_element_type=F32), 0.0)
    out_ref[...] = jnp.dot(t, w2_ref[...], preferred_element_type=F32) + b2_ref[...]


def _node_body_fin(q_ref, agg_ref, wa_ref, w2_ref, b2_ref, xorig_ref, out_ref):
    agg = agg_ref[0] + agg_ref[1]
    t = jnp.maximum(q_ref[...] + jnp.dot(agg, wa_ref[...],
                                         preferred_element_type=F32), 0.0)
    xo = jnp.dot(t, w2_ref[...], preferred_element_type=F32) + b2_ref[...]
    out_ref[...] = 0.5 * xorig_ref[...] + 0.5 * xo


def _node_mlp(q, agg2, wa, w2, b2, xorig):
    args = [q, agg2, wa, w2, b2.reshape(1, ND)]
    if xorig is None:
        return pl.pallas_call(
            _node_body_mid,
            out_shape=jax.ShapeDtypeStruct((N, ND), F32),
        )(*args)
    return pl.pallas_call(
        _node_body_fin,
        out_shape=jax.ShapeDtypeStruct((N, ND), F32),
    )(*(args + [xorig]))


GNB = 2
GNC = BPW // GCH


def _sc_gather(ps, pd, src, dst):
    mesh = plsc.VectorSubcoreMesh(core_axis_name="c", subcore_axis_name="s")

    @functools.partial(
        pl.kernel, mesh=mesh,
        out_type=jax.ShapeDtypeStruct((E, DP), F32),
        compiler_params=pltpu.CompilerParams(use_tc_tiling_on_sc=False),
        scratch_types=[
            pltpu.VMEM((BPW,), jnp.int32),
            pltpu.VMEM((BPW,), jnp.int32),
        ] + [pltpu.VMEM((GCH, DP), F32) for _ in range(2 * GNB)]
          + [pltpu.SemaphoreType.DMA for _ in range(2 * GNB)],
    )
    def k(ps_hbm, pd_hbm, src_hbm, dst_hbm, out_hbm, si_all, di_all, *bufs):
        rs = bufs[0:GNB]
        rd = bufs[GNB:2 * GNB]
        sg = bufs[2 * GNB:3 * GNB]
        ss = bufs[3 * GNB:4 * GNB]
        wid = lax.axis_index("s") * NC + lax.axis_index("c")
        base = wid * BPW
        pltpu.sync_copy(src_hbm.at[pl.ds(base, BPW)], si_all)
        pltpu.sync_copy(dst_hbm.at[pl.ds(base, BPW)], di_all)

        def issue_gather(j, b):
            loc = j * GCH
            c1 = pltpu.async_copy(ps_hbm.at[si_all.at[pl.ds(loc, GCH)]],
                                  rs[b], sg[b])
            c2 = pltpu.async_copy(pd_hbm.at[di_all.at[pl.ds(loc, GCH)]],
                                  rd[b], sg[b])
            return c1, c2

        inflight = [None] * GNB
        stores = [None] * GNB
        for j in range(min(GNB, GNC)):
            inflight[j % GNB] = issue_gather(j, j % GNB)
        for j in range(GNC):
            b = j % GNB
            c1, c2 = inflight[b]
            c1.wait()
            c2.wait()

            def addrow(r, cc):
                for c in range(ADD_LANES // 16):
                    sl = (r, pl.ds(c * 16, 16))
                    rs[b][sl] = rs[b][sl] + rd[b][sl]
                return cc

            lax.fori_loop(0, GCH, addrow, 0)
            stores[b] = pltpu.async_copy(
                rs[b], out_hbm.at[pl.ds(base + j * GCH, GCH)], ss[b])
            if j + GNB < GNC:
                stores[b].wait()
                inflight[b] = issue_gather(j + GNB, b)
        for j in range(max(GNC - GNB, 0), GNC):
            stores[j % GNB].wait()

    return k(ps, pd, src, dst)


SNB = 2
SNC = BPW // SCH


def _sc_scatter(eo, dst):
    mesh = plsc.VectorSubcoreMesh(core_axis_name="c", subcore_axis_name="s")
    rpt = N // NS

    @functools.partial(
        pl.kernel, mesh=mesh,
        out_type=jax.ShapeDtypeStruct((NC, N, ED), F32),
        compiler_params=pltpu.CompilerParams(use_tc_tiling_on_sc=False),
        scratch_types=[
            pltpu.VMEM((SNC, SCH), jnp.int32),
            pltpu.VMEM((rpt, ED), F32),
            pltpu.VMEM_SHARED((N, ED), F32),
        ] + [pltpu.VMEM((SCH, ED), F32) for _ in range(SNB)]
          + [pltpu.SemaphoreType.DMA for _ in range(SNB)],
    )
    def k(eo_hbm, dst_hbm, out_hbm, dstw, stage, aggsh, *bufs):
        rows = bufs[0:SNB]
        sr = bufs[SNB:2 * SNB]
        cid = lax.axis_index("c")
        sid = lax.axis_index("s")
        wid = sid * NC + cid
        base = wid * BPW
        pltpu.sync_copy(dst_hbm.at[pl.ds(wid * SNC, SNC)], dstw)

        def zrow(r, carry):
            stage[r, pl.ds(0, ED)] = jnp.zeros((ED,), F32)
            return carry

        lax.fori_loop(0, rpt, zrow, 0)
        pltpu.sync_copy(stage, aggsh.at[pl.ds(sid * rpt, rpt)])
        plsc.subcore_barrier()

        for b in range(SNB):
            pltpu.async_copy(eo_hbm.at[pl.ds(base + b * SCH, SCH)],
                             rows[b], sr[b])

        def outer(jj, carry):
            for b in range(SNB):
                j = jj * SNB + b
                pltpu.make_async_copy(eo_hbm.at[pl.ds(0, SCH)],
                                      rows[b], sr[b]).wait()
                pltpu.sync_copy(rows[b], aggsh.at[dstw.at[j]], add=True)

                @pl.when(j + SNB < SNC)
                def _():
                    pltpu.async_copy(
                        eo_hbm.at[pl.ds(base + (j + SNB) * SCH, SCH)],
                        rows[b], sr[b])
            return carry

        lax.fori_loop(0, SNC // SNB, outer, 0)
        plsc.subcore_barrier()
        pltpu.sync_copy(aggsh.at[pl.ds(sid * rpt, rpt)], stage)
        pltpu.sync_copy(stage, out_hbm.at[cid, pl.ds(sid * rpt, rpt)])

    return k(eo, dst)


def kernel(x, edge_index, edge_attr, params):
    layers = params["layers"]
    src = edge_index[0]
    dst = edge_index[1]
    cur_x, cur_e = x, edge_attr
    out_x = out_e = None
    for li, p in enumerate(layers):
        final = li == len(layers) - 1
        pe, pn = p["edge_mlp"], p["node_mlp"]
        w1 = pe["W1"]
        pad = ((0, 0), (0, DP - HID))
        w1s = jnp.pad(w1[:ND], pad)
        w1d = jnp.pad(w1[ND:2 * ND], pad)
        w1e = w1[2 * ND:]
        wn1 = pn["W1"]
        ps, pd_, q = _node_prep(cur_x, p["bn_node"]["gamma"], p["bn_node"]["beta"],
                                w1s, w1d, wn1[:ND], pn["b1"])
        stats = _colstats(cur_e)
        g = _sc_gather(ps, pd_, src, dst)
        if final:
            eo, out_e = _edge_mlp(g, cur_e, stats, p["bn_edge"]["gamma"],
                                  p["bn_edge"]["beta"], w1e, pe["b1"], pe["W2"],
                                  pe["b2"], edge_attr)
        else:
            eo = _edge_mlp(g, cur_e, stats, p["bn_edge"]["gamma"],
                           p["bn_edge"]["beta"], w1e, pe["b1"], pe["W2"],
                           pe["b2"], None)
        agg2 = _sc_scatter(eo, dst.reshape(E // SCH, SCH))
        xo = _node_mlp(q, agg2, wn1[ND:], pn["W2"], pn["b2"],
                       x if final else None)
        if final:
            out_x = xo
        cur_x, cur_e = xo, eo
    return (out_x, out_e)
